# Initial kernel scaffold; baseline (speedup 1.0000x reference)
import jax
import jax.numpy as jnp
from jax.experimental import pallas as pl


def kernel(src, type_input, word_table, pos_table, seg_table, ln_w, ln_b):
    raise NotImplementedError("write your pallas kernel here")


# speedup vs baseline: 4.8406x; 4.8406x over previous
import functools

import jax
import jax.numpy as jnp
from jax import lax
from jax.experimental import pallas as pl
from jax.experimental.pallas import tpu as pltpu
from jax.experimental.pallas import tpu_sc as plsc

VOCAB = 1000000
DIM = 128
B = 1024
L = 200
NC = 2
NS = 16
NW = NC * NS
SEQ_PER_W = B // NW
NG = DIM // 16
C0 = 128
C1 = L - C0
TOK_UNROLL = 2
EPS = 1e-5


def _permute(v, idx):
    dnums = lax.GatherDimensionNumbers(
        offset_dims=(), collapsed_slice_dims=(0,), start_index_map=(0,))
    return lax.gather(v, idx[:, None], dnums, slice_sizes=(1,),
                      mode=lax.GatherScatterMode.PROMISE_IN_BOUNDS)


def _allsum(v):
    lanes = jnp.arange(16, dtype=jnp.int32)
    for sh in (8, 4, 2, 1):
        v = v + _permute(v, lanes ^ sh)
    return v


def _rsqrt(x):
    i = plsc.bitcast(x, jnp.int32)
    y = plsc.bitcast(jnp.int32(0x5F3759DF) - (i >> 1), jnp.float32)
    for _ in range(3):
        y = y * (1.5 - 0.5 * x * y * y)
    return y


def _emb_body(src_hbm, type_hbm, word_hbm, pos_hbm, seg_hbm, lnw_hbm, lnb_hbm,
              out_hbm, comb_v, rows0, rows1, idxA0, idxB0, idxA1, idxB1,
              typ0, typ1, seg_v, lnw_v, lnb_v,
              gsem0, gsem1, isem0, isem1, osem0, osem1, tsem0, tsem1):
    wid = lax.axis_index("c") * NS + lax.axis_index("s")

    for s in range(3):
        pltpu.sync_copy(pos_hbm.at[pl.ds(0, L)], comb_v.at[pl.ds(s * L, L)])
    pltpu.sync_copy(seg_hbm, seg_v)
    pltpu.sync_copy(lnw_hbm, lnw_v)
    pltpu.sync_copy(lnb_hbm, lnb_v)
    segr = [[seg_v[s, pl.ds(g * 16, 16)] for g in range(NG)] for s in range(3)]

    @plsc.parallel_loop(0, L, unroll=2)
    def _comb_loop(p):
        for s in range(3):
            for g in range(NG):
                sl = pl.ds(g * 16, 16)
                comb_v[s * L + p, sl] = comb_v[s * L + p, sl] + segr[s][g]

    wreg = [lnw_v[pl.ds(g * 16, 16)] for g in range(NG)]
    breg = [lnb_v[pl.ds(g * 16, 16)] for g in range(NG)]

    idxA = [idxA0, idxA1]
    idxB = [idxB0, idxB1]
    typ_v = [typ0, typ1]
    rows_v = [rows0, rows1]
    isem = [isem0, isem1]
    gsem = [gsem0, gsem1]
    osem = [osem0, osem1]
    tsem = [tsem0, tsem1]

    def seq_of(j):
        return wid * SEQ_PER_W + j

    def idx_copies(j, sl):
        base = seq_of(j) * L
        return [pltpu.make_async_copy(src_hbm.at[pl.ds(base, C0)],
                                      idxA[sl], isem[sl]),
                pltpu.make_async_copy(src_hbm.at[pl.ds(base + C0, C1)],
                                      idxB[sl], isem[sl])]

    def typ_copy(j, sl):
        base = seq_of(j) * L
        return pltpu.make_async_copy(type_hbm.at[pl.ds(base, L)],
                                     typ_v[sl].at[pl.ds(0, L)], tsem[sl])

    def gather_copies(sl):
        return [pltpu.make_async_copy(word_hbm.at[idxA[sl]],
                                      rows_v[sl].at[pl.ds(0, C0)], gsem[sl]),
                pltpu.make_async_copy(word_hbm.at[idxB[sl]],
                                      rows_v[sl].at[pl.ds(C0, C1)], gsem[sl])]

    def out_copy(j, sl):
        return pltpu.make_async_copy(rows_v[sl], out_hbm.at[seq_of(j)],
                                     osem[sl])

    def compute(rows, typ):
        @plsc.parallel_loop(0, L, unroll=TOK_UNROLL)
        def _tok(t):
            s = typ[pl.ds(t, 16)][0]
            r = s * L + t
            x = [rows[t, pl.ds(g * 16, 16)] + comb_v[r, pl.ds(g * 16, 16)]
                 for g in range(NG)]
            ssum = (((x[0] + x[1]) + (x[2] + x[3]))
                    + ((x[4] + x[5]) + (x[6] + x[7])))
            qsum = (((x[0] * x[0] + x[1] * x[1])
                     + (x[2] * x[2] + x[3] * x[3]))
                    + ((x[4] * x[4] + x[5] * x[5])
                       + (x[6] * x[6] + x[7] * x[7])))
            sb = _allsum(ssum)
            qb = _allsum(qsum)
            mu = sb * (1.0 / DIM)
            var = qb * (1.0 / DIM) - mu * mu
            rs = _rsqrt(var + EPS)
            for g in range(NG):
                rows[t, pl.ds(g * 16, 16)] = ((x[g] - mu) * rs * wreg[g]
                                              + breg[g])

    for c in idx_copies(0, 0):
        c.start()
    for c in idx_copies(1, 1):
        c.start()
    typ_copy(0, 0).start()
    typ_copy(1, 1).start()
    for c in idx_copies(0, 0):
        c.wait()
    for c in gather_copies(0):
        c.start()

    @pl.loop(0, SEQ_PER_W, step=2)
    def _seq_loop(j0):
        for dj in range(2):
            j = j0 + dj
            sl = dj
            nsl = 1 - dj
            for c in gather_copies(sl):
                c.wait()

            @pl.when(j + 2 < SEQ_PER_W)
            def _():
                for c in idx_copies(j + 2, sl):
                    c.start()

            @pl.when(j + 1 < SEQ_PER_W)
            def _():
                for c in idx_copies(j + 1, nsl):
                    c.wait()

                @pl.when(j >= 1)
                def _():
                    out_copy(j - 1, nsl).wait()

                for c in gather_copies(nsl):
                    c.start()

            typ_copy(j, sl).wait()
            compute(rows_v[sl], typ_v[sl])

            @pl.when(j + 2 < SEQ_PER_W)
            def _():
                typ_copy(j + 2, sl).start()

            out_copy(j, sl).start()

    out_copy(SEQ_PER_W - 2, 0).wait()
    out_copy(SEQ_PER_W - 1, 1).wait()


_SCRATCH = [
        pltpu.VMEM((3 * L, DIM), jnp.float32),
        pltpu.VMEM((L, DIM), jnp.float32),
        pltpu.VMEM((L, DIM), jnp.float32),
        pltpu.VMEM((C0,), jnp.int32),
        pltpu.VMEM((C1,), jnp.int32),
        pltpu.VMEM((C0,), jnp.int32),
        pltpu.VMEM((C1,), jnp.int32),
        pltpu.VMEM((L + 16,), jnp.int32),
        pltpu.VMEM((L + 16,), jnp.int32),
        pltpu.VMEM((3, DIM), jnp.float32),
        pltpu.VMEM((DIM,), jnp.float32),
        pltpu.VMEM((DIM,), jnp.float32),
        pltpu.SemaphoreType.DMA,
        pltpu.SemaphoreType.DMA,
        pltpu.SemaphoreType.DMA,
        pltpu.SemaphoreType.DMA,
        pltpu.SemaphoreType.DMA,
        pltpu.SemaphoreType.DMA,
        pltpu.SemaphoreType.DMA,
        pltpu.SemaphoreType.DMA,
]


@functools.cache
def _emb_kernel():
    return pl.kernel(
        _emb_body,
        out_type=jax.ShapeDtypeStruct((B, L, DIM), jnp.float32),
        mesh=plsc.VectorSubcoreMesh(core_axis_name="c", subcore_axis_name="s"),
        compiler_params=pltpu.CompilerParams(needs_layout_passes=False),
        scratch_types=_SCRATCH,
    )


def kernel(src, type_input, word_table, pos_table, seg_table, ln_w, ln_b):
    src_i = src.astype(jnp.int32).reshape(-1)
    typ_i = type_input.astype(jnp.int32).reshape(-1)
    return _emb_kernel()(src_i, typ_i, word_table, pos_table, seg_table,
                         ln_w, ln_b)


# speedup vs baseline: 4.9272x; 1.0179x over previous
import functools

import jax
import jax.numpy as jnp
from jax import lax
from jax.experimental import pallas as pl
from jax.experimental.pallas import tpu as pltpu
from jax.experimental.pallas import tpu_sc as plsc

VOCAB = 1000000
DIM = 128
B = 1024
L = 200
NC = 2
NS = 16
NW = NC * NS
SEQ_PER_W = B // NW
NG = DIM // 16
C0 = 128
C1 = L - C0
TOK_UNROLL = 2
EPS = 1e-5


def _permute(v, idx):
    dnums = lax.GatherDimensionNumbers(
        offset_dims=(), collapsed_slice_dims=(0,), start_index_map=(0,))
    return lax.gather(v, idx[:, None], dnums, slice_sizes=(1,),
                      mode=lax.GatherScatterMode.PROMISE_IN_BOUNDS)


def _allsum(v):
    lanes = jnp.arange(16, dtype=jnp.int32)
    for sh in (8, 4, 2, 1):
        v = v + _permute(v, lanes ^ sh)
    return v


def _rsqrt(x):
    i = plsc.bitcast(x, jnp.int32)
    y = plsc.bitcast(jnp.int32(0x5F3759DF) - (i >> 1), jnp.float32)
    for _ in range(2):
        y = y * (1.5 - 0.5 * x * y * y)
    return y


def _emb_body(src_hbm, type_hbm, word_hbm, pos_hbm, seg_hbm, lnw_hbm, lnb_hbm,
              out_hbm, comb_v, rows0, rows1, idxA0, idxB0, idxA1, idxB1,
              typ0, typ1, seg_v, lnw_v, lnb_v,
              gsem0, gsem1, isem0, isem1, osem0, osem1, tsem0, tsem1):
    wid = lax.axis_index("c") * NS + lax.axis_index("s")

    for s in range(3):
        pltpu.sync_copy(pos_hbm.at[pl.ds(0, L)], comb_v.at[pl.ds(s * L, L)])
    pltpu.sync_copy(seg_hbm, seg_v)
    pltpu.sync_copy(lnw_hbm, lnw_v)
    pltpu.sync_copy(lnb_hbm, lnb_v)
    segr = [[seg_v[s, pl.ds(g * 16, 16)] for g in range(NG)] for s in range(3)]

    @plsc.parallel_loop(0, L, unroll=2)
    def _comb_loop(p):
        for s in range(3):
            for g in range(NG):
                sl = pl.ds(g * 16, 16)
                comb_v[s * L + p, sl] = comb_v[s * L + p, sl] + segr[s][g]

    wreg = [lnw_v[pl.ds(g * 16, 16)] for g in range(NG)]
    breg = [lnb_v[pl.ds(g * 16, 16)] for g in range(NG)]

    idxA = [idxA0, idxA1]
    idxB = [idxB0, idxB1]
    typ_v = [typ0, typ1]
    rows_v = [rows0, rows1]
    isem = [isem0, isem1]
    gsem = [gsem0, gsem1]
    osem = [osem0, osem1]
    tsem = [tsem0, tsem1]

    def seq_of(j):
        return wid * SEQ_PER_W + j

    def idx_copies(j, sl):
        base = seq_of(j) * L
        return [pltpu.make_async_copy(src_hbm.at[pl.ds(base, C0)],
                                      idxA[sl], isem[sl]),
                pltpu.make_async_copy(src_hbm.at[pl.ds(base + C0, C1)],
                                      idxB[sl], isem[sl])]

    def typ_copy(j, sl):
        base = seq_of(j) * L
        return pltpu.make_async_copy(type_hbm.at[pl.ds(base, L)],
                                     typ_v[sl].at[pl.ds(0, L)], tsem[sl])

    def gather_copies(sl):
        return [pltpu.make_async_copy(word_hbm.at[idxA[sl]],
                                      rows_v[sl].at[pl.ds(0, C0)], gsem[sl]),
                pltpu.make_async_copy(word_hbm.at[idxB[sl]],
                                      rows_v[sl].at[pl.ds(C0, C1)], gsem[sl])]

    def out_copy(j, sl):
        return pltpu.make_async_copy(rows_v[sl], out_hbm.at[seq_of(j)],
                                     osem[sl])

    def compute(rows, typ):
        @plsc.parallel_loop(0, L, unroll=TOK_UNROLL)
        def _tok(t):
            s = typ[pl.ds(t, 16)][0]
            r = s * L + t
            x = [rows[t, pl.ds(g * 16, 16)] + comb_v[r, pl.ds(g * 16, 16)]
                 for g in range(NG)]
            ssum = (((x[0] + x[1]) + (x[2] + x[3]))
                    + ((x[4] + x[5]) + (x[6] + x[7])))
            qsum = (((x[0] * x[0] + x[1] * x[1])
                     + (x[2] * x[2] + x[3] * x[3]))
                    + ((x[4] * x[4] + x[5] * x[5])
                       + (x[6] * x[6] + x[7] * x[7])))
            sb = _allsum(ssum)
            qb = _allsum(qsum)
            mu = sb * (1.0 / DIM)
            var = qb * (1.0 / DIM) - mu * mu
            rs = _rsqrt(var + EPS)
            for g in range(NG):
                rows[t, pl.ds(g * 16, 16)] = ((x[g] - mu) * rs * wreg[g]
                                              + breg[g])

    for c in idx_copies(0, 0):
        c.start()
    for c in idx_copies(1, 1):
        c.start()
    typ_copy(0, 0).start()
    typ_copy(1, 1).start()
    for c in idx_copies(0, 0):
        c.wait()
    for c in gather_copies(0):
        c.start()

    @pl.loop(0, SEQ_PER_W, step=2)
    def _seq_loop(j0):
        for dj in range(2):
            j = j0 + dj
            sl = dj
            nsl = 1 - dj
            for c in gather_copies(sl):
                c.wait()

            @pl.when(j + 2 < SEQ_PER_W)
            def _():
                for c in idx_copies(j + 2, sl):
                    c.start()

            @pl.when(j + 1 < SEQ_PER_W)
            def _():
                for c in idx_copies(j + 1, nsl):
                    c.wait()

                @pl.when(j >= 1)
                def _():
                    out_copy(j - 1, nsl).wait()

                for c in gather_copies(nsl):
                    c.start()

            typ_copy(j, sl).wait()
            compute(rows_v[sl], typ_v[sl])

            @pl.when(j + 2 < SEQ_PER_W)
            def _():
                typ_copy(j + 2, sl).start()

            out_copy(j, sl).start()

    out_copy(SEQ_PER_W - 2, 0).wait()
    out_copy(SEQ_PER_W - 1, 1).wait()


_SCRATCH = [
        pltpu.VMEM((3 * L, DIM), jnp.float32),
        pltpu.VMEM((L, DIM), jnp.float32),
        pltpu.VMEM((L, DIM), jnp.float32),
        pltpu.VMEM((C0,), jnp.int32),
        pltpu.VMEM((C1,), jnp.int32),
        pltpu.VMEM((C0,), jnp.int32),
        pltpu.VMEM((C1,), jnp.int32),
        pltpu.VMEM((L + 16,), jnp.int32),
        pltpu.VMEM((L + 16,), jnp.int32),
        pltpu.VMEM((3, DIM), jnp.float32),
        pltpu.VMEM((DIM,), jnp.float32),
        pltpu.VMEM((DIM,), jnp.float32),
        pltpu.SemaphoreType.DMA,
        pltpu.SemaphoreType.DMA,
        pltpu.SemaphoreType.DMA,
        pltpu.SemaphoreType.DMA,
        pltpu.SemaphoreType.DMA,
        pltpu.SemaphoreType.DMA,
        pltpu.SemaphoreType.DMA,
        pltpu.SemaphoreType.DMA,
]


@functools.cache
def _emb_kernel():
    return pl.kernel(
        _emb_body,
        out_type=jax.ShapeDtypeStruct((B, L, DIM), jnp.float32),
        mesh=plsc.VectorSubcoreMesh(core_axis_name="c", subcore_axis_name="s"),
        compiler_params=pltpu.CompilerParams(needs_layout_passes=False),
        scratch_types=_SCRATCH,
    )


def kernel(src, type_input, word_table, pos_table, seg_table, ln_w, ln_b):
    src_i = src.astype(jnp.int32).reshape(-1)
    typ_i = type_input.astype(jnp.int32).reshape(-1)
    return _emb_kernel()(src_i, typ_i, word_table, pos_table, seg_table,
                         ln_w, ln_b)
